# Initial kernel scaffold; baseline (speedup 1.0000x reference)
import jax
import jax.numpy as jnp
from jax.experimental import pallas as pl


def kernel(xyz, feat, W_pre, W1_l1, Ws_l1, Wm_l1, W1_l2, Ws_l2, Wm_l2, W1_l3, Ws_l3, Wm_l3, W_up2, W_up1, W_h1, b_h1, W_h2, b_h2):
    raise NotImplementedError("write your pallas kernel here")


# speedup vs baseline: 3.7835x; 3.7835x over previous
import functools
import math

import jax
import jax.numpy as jnp
from jax import lax
from jax.experimental import pallas as pl

_IT = False
_BINV = 1.0 / math.sqrt(1.0 + 1e-5)
_K = 16


def _lrelu(x):
    return jnp.where(x >= 0, x, 0.1 * x)


def _bq(x):
    return x.astype(jnp.bfloat16).astype(jnp.float32)


def _knn_body(q_ref, p_ref, i_ref, *, nk, kpad, npts):
    q = q_ref[0]
    p = p_ref[0]
    qn = jnp.sum(q * q, axis=1)[:, None]
    pn = jnp.sum(p * p, axis=1)[None, :]
    dots_b = lax.dot_general(q.astype(jnp.bfloat16), p.astype(jnp.bfloat16),
                             (((1,), (1,)), ((), ())),
                             preferred_element_type=jnp.float32)
    dsel = jnp.sqrt(jnp.maximum(qn + pn - 2.0 * dots_b, 0.0))
    r = q.shape[0]
    cols = lax.broadcasted_iota(jnp.int32, (r, npts), 1)
    ki = lax.broadcasted_iota(jnp.int32, (r, kpad), 1)
    idx_acc = jnp.zeros((r, kpad), jnp.int32)
    big = jnp.float32(3.0e38)
    for j in range(nk):
        m = jnp.min(dsel, axis=1, keepdims=True)
        cand = jnp.where(dsel <= m, cols, npts)
        im = jnp.min(cand, axis=1, keepdims=True)
        idx_acc = jnp.where(ki == j, im, idx_acc)
        if j < nk - 1:
            dsel = jnp.where(cols == im, big, dsel)
    i_ref[0] = idx_acc


def _knn(xyz_q, xyz_p, nk, row_block=256):
    b, nq, _ = xyz_q.shape
    npts = xyz_p.shape[1]
    kpad = max(nk, 8)
    body = functools.partial(_knn_body, nk=nk, kpad=kpad, npts=npts)
    return pl.pallas_call(
        body,
        grid=(b, nq // row_block),
        in_specs=[
            pl.BlockSpec((1, row_block, 3), lambda bb, i: (bb, i, 0)),
            pl.BlockSpec((1, npts, 3), lambda bb, i: (bb, 0, 0)),
        ],
        out_specs=pl.BlockSpec((1, row_block, kpad), lambda bb, i: (bb, i, 0)),
        out_shape=jax.ShapeDtypeStruct((b, nq, kpad), jnp.int32),
        interpret=_IT,
    )(xyz_q, xyz_p)


def _lfa_body(cen_ref, g_ref, nx_ref, ws_ref, wm_ref, wr_ref, o_ref):
    cen = cen_ref[...]
    g = g_ref[...]
    nx = nx_ref[...]
    wr = _bq(wr_ref[...])
    diff = cen[None] - nx
    d0 = diff[:, :, 0]
    d1 = diff[:, :, 1]
    d2c = diff[:, :, 2]
    dd = d0 * d0 + d1 * d1 + d2c * d2c
    dist = jnp.sqrt(jnp.maximum(dd, 1e-20))
    kk, pp, cc = g.shape
    cenb = jnp.broadcast_to(cen[None], (kk, pp, 3))
    rels = (d0, d1, d2c, dist,
            cenb[:, :, 0], cenb[:, :, 1], cenb[:, :, 2],
            nx[:, :, 0], nx[:, :, 1], nx[:, :, 2])
    x = g
    for c, rel in enumerate(rels):
        x = x + _bq(rel)[:, :, None] * wr[c][None, None, :]
    x = _lrelu(_BINV * x)
    s = (x.reshape(kk * pp, cc) @ ws_ref[...]).reshape(kk, pp, cc)
    m = jnp.max(s, axis=0)
    e = jnp.exp(s - m[None])
    den = jnp.sum(e, axis=0)
    pooled = jnp.sum(x * e, axis=0) / den
    o_ref[...] = _lrelu(_BINV * (pooled @ wm_ref[...]))


def _lfa(cen, g3, nx3, ws_t, wm_t, wr, row_block=128):
    bn, c = g3.shape[1], g3.shape[2]
    return pl.pallas_call(
        _lfa_body,
        grid=(bn // row_block,),
        in_specs=[
            pl.BlockSpec((row_block, 3), lambda i: (i, 0)),
            pl.BlockSpec((_K, row_block, c), lambda i: (0, i, 0)),
            pl.BlockSpec((_K, row_block, 3), lambda i: (0, i, 0)),
            pl.BlockSpec((c, c), lambda i: (0, 0)),
            pl.BlockSpec((c, c), lambda i: (0, 0)),
            pl.BlockSpec((16, c), lambda i: (0, 0)),
        ],
        out_specs=pl.BlockSpec((row_block, c), lambda i: (i, 0)),
        out_shape=jax.ShapeDtypeStruct((bn, c), jnp.float32),
        interpret=_IT,
    )(cen, g3, nx3, ws_t, wm_t, wr)


def _k0_body(f_ref, wpre_ref, wf_ref, g_ref):
    x0 = _lrelu(_BINV * (f_ref[...] @ wpre_ref[...]))
    g_ref[...] = x0 @ wf_ref[...]


def _k0(feat2, wpre_t, wf_t, row_block=512):
    bn, fin = feat2.shape
    c = wf_t.shape[1]
    return pl.pallas_call(
        _k0_body,
        grid=(bn // row_block,),
        in_specs=[
            pl.BlockSpec((row_block, fin), lambda i: (i, 0)),
            pl.BlockSpec((fin, c), lambda i: (0, 0)),
            pl.BlockSpec((c, c), lambda i: (0, 0)),
        ],
        out_specs=pl.BlockSpec((row_block, c), lambda i: (i, 0)),
        out_shape=jax.ShapeDtypeStruct((bn, c), jnp.float32),
        interpret=_IT,
    )(feat2, wpre_t, wf_t)


def _kproj_body(x_ref, wf_ref, wp_ref, g_ref, p_ref):
    x = x_ref[...]
    g_ref[...] = x @ wf_ref[...]
    p_ref[...] = x @ wp_ref[...]


def _kproj(xf, wf_t, wp_t, row_block=512):
    bn, fin = xf.shape
    c = wf_t.shape[1]
    cp = wp_t.shape[1]
    return pl.pallas_call(
        _kproj_body,
        grid=(bn // row_block,),
        in_specs=[
            pl.BlockSpec((row_block, fin), lambda i: (i, 0)),
            pl.BlockSpec((fin, c), lambda i: (0, 0)),
            pl.BlockSpec((fin, cp), lambda i: (0, 0)),
        ],
        out_specs=[
            pl.BlockSpec((row_block, c), lambda i: (i, 0)),
            pl.BlockSpec((row_block, cp), lambda i: (i, 0)),
        ],
        out_shape=[
            jax.ShapeDtypeStruct((bn, c), jnp.float32),
            jax.ShapeDtypeStruct((bn, cp), jnp.float32),
        ],
        interpret=_IT,
    )(xf, wf_t, wp_t)


def _kmm_body(x_ref, w_ref, o_ref):
    o_ref[...] = x_ref[...] @ w_ref[...]


def _kmm(x, w_t, row_block=512):
    bn, fin = x.shape
    c = w_t.shape[1]
    return pl.pallas_call(
        _kmm_body,
        grid=(bn // row_block,),
        in_specs=[
            pl.BlockSpec((row_block, fin), lambda i: (i, 0)),
            pl.BlockSpec((fin, c), lambda i: (0, 0)),
        ],
        out_specs=pl.BlockSpec((row_block, c), lambda i: (i, 0)),
        out_shape=jax.ShapeDtypeStruct((bn, c), jnp.float32),
        interpret=_IT,
    )(x, w_t)


def _kmid_body(u_ref, p_ref, w_ref, o_ref):
    t = _lrelu(_BINV * (u_ref[...] + p_ref[...]))
    o_ref[...] = t @ w_ref[...]


def _kmid(u2, p2, w_t, row_block=512):
    bn, fin = u2.shape
    c = w_t.shape[1]
    return pl.pallas_call(
        _kmid_body,
        grid=(bn // row_block,),
        in_specs=[
            pl.BlockSpec((row_block, fin), lambda i: (i, 0)),
            pl.BlockSpec((row_block, fin), lambda i: (i, 0)),
            pl.BlockSpec((fin, c), lambda i: (0, 0)),
        ],
        out_specs=pl.BlockSpec((row_block, c), lambda i: (i, 0)),
        out_shape=jax.ShapeDtypeStruct((bn, c), jnp.float32),
        interpret=_IT,
    )(u2, p2, w_t)


def _ktail_body(u_ref, p_ref, w1_ref, b1_ref, w2_ref, b2_ref, o_ref):
    d1 = _lrelu(_BINV * (u_ref[...] + p_ref[...]))
    h = _lrelu(d1 @ w1_ref[...] + b1_ref[...])
    o_ref[...] = h @ w2_ref[...] + b2_ref[...]


def _ktail(u1, p1, w1_t, b1, w2_t, b2, row_block=512):
    bn, fin = u1.shape
    ch = w1_t.shape[1]
    co = w2_t.shape[1]
    return pl.pallas_call(
        _ktail_body,
        grid=(bn // row_block,),
        in_specs=[
            pl.BlockSpec((row_block, fin), lambda i: (i, 0)),
            pl.BlockSpec((row_block, fin), lambda i: (i, 0)),
            pl.BlockSpec((fin, ch), lambda i: (0, 0)),
            pl.BlockSpec((1, ch), lambda i: (0, 0)),
            pl.BlockSpec((ch, co), lambda i: (0, 0)),
            pl.BlockSpec((1, co), lambda i: (0, 0)),
        ],
        out_specs=pl.BlockSpec((row_block, co), lambda i: (i, 0)),
        out_shape=jax.ShapeDtypeStruct((bn, co), jnp.float32),
        interpret=_IT,
    )(u1, p1, w1_t, b1, w2_t, b2)


def _gather_rows(table, idx_flat):
    return jnp.take(table, idx_flat, axis=0)


def kernel(xyz, feat, W_pre, W1_l1, Ws_l1, Wm_l1, W1_l2, Ws_l2, Wm_l2,
           W1_l3, Ws_l3, Wm_l3, W_up2, W_up1, W_h1, b_h1, W_h2, b_h2):
    b, n, _ = xyz.shape
    nfeat = feat.shape[-1]
    ncls = W_h2.shape[0]
    m1 = max(32, -(-n // 2))
    m2 = max(32, -(-m1 // 2))
    bn = b * n

    xyz2 = xyz.reshape(bn, 3)
    feat2 = feat.reshape(bn, nfeat)
    offn = (jnp.arange(b, dtype=jnp.int32) * n)[:, None]
    offm1 = (jnp.arange(b, dtype=jnp.int32) * m1)[:, None]
    offm2 = (jnp.arange(b, dtype=jnp.int32) * m2)[:, None]

    def wr_pad(w1, f):
        return jnp.zeros((16, w1.shape[0]), jnp.float32).at[:10].set(
            w1[:, f:f + 10].T)

    g1t = _k0(feat2, W_pre.T, W1_l1[:, :64].T)
    n0 = _knn(xyz, xyz, _K)
    idx0t = (n0[..., :_K].reshape(b, n * _K) + offn).reshape(bn, _K).T
    idx0f = idx0t.reshape(-1)
    g1 = _gather_rows(g1t, idx0f).reshape(_K, bn, 64)
    nx0 = _gather_rows(xyz2, idx0f).reshape(_K, bn, 3)
    x1 = _lfa(xyz2, g1, nx0, Ws_l1.T, Wm_l1.T, wr_pad(W1_l1, 64))

    sel1 = jax.random.randint(jax.random.key(1), (b, m1), 0, n)
    xyz1 = jnp.take_along_axis(xyz, sel1[:, :, None], axis=1)
    xyz1f = xyz1.reshape(b * m1, 3)

    g2t, p1 = _kproj(x1, W1_l2[:, :64].T, W_up1[:, 128:].T)
    n1 = _knn(xyz1, xyz1, _K)
    n1r = n1[..., :_K].reshape(b, m1 * _K)
    comp1 = (jnp.take_along_axis(sel1, n1r, axis=1) + offn)
    idx1t = comp1.reshape(b * m1, _K).T.reshape(-1)
    nb1t = (n1r + offm1).reshape(b * m1, _K).T.reshape(-1)
    g2 = _gather_rows(g2t, idx1t).reshape(_K, b * m1, 128)
    nx1 = _gather_rows(xyz1f, nb1t).reshape(_K, b * m1, 3)
    x2 = _lfa(xyz1f, g2, nx1, Ws_l2.T, Wm_l2.T, wr_pad(W1_l2, 64))

    sel2 = jax.random.randint(jax.random.key(2), (b, m2), 0, m1)
    xyz2d = jnp.take_along_axis(xyz1, sel2[:, :, None], axis=1)
    xyz2df = xyz2d.reshape(b * m2, 3)

    g3t, p2 = _kproj(x2, W1_l3[:, :128].T, W_up2[:, 256:].T)
    n2 = _knn(xyz2d, xyz2d, _K)
    n2r = n2[..., :_K].reshape(b, m2 * _K)
    comp2 = (jnp.take_along_axis(sel2, n2r, axis=1) + offm1)
    idx2t = comp2.reshape(b * m2, _K).T.reshape(-1)
    nb2t = (n2r + offm2).reshape(b * m2, _K).T.reshape(-1)
    g3 = _gather_rows(g3t, idx2t).reshape(_K, b * m2, 256)
    nx2 = _gather_rows(xyz2df, nb2t).reshape(_K, b * m2, 3)
    x3 = _lfa(xyz2df, g3, nx2, Ws_l3.T, Wm_l3.T, wr_pad(W1_l3, 128))

    p3 = _kmm(x3, W_up2[:, :256].T)
    nn2 = _knn(xyz1, xyz2d, 1)
    u2 = _gather_rows(p3, (nn2[..., 0] + offm2).reshape(-1))
    p2d = _kmid(u2, p2, W_up1[:, :128].T)
    nn1 = _knn(xyz, xyz1, 1)
    u1 = _gather_rows(p2d, (nn1[..., 0] + offm1).reshape(-1))

    w2_pad = jnp.zeros((64, 128), jnp.float32).at[:, :ncls].set(W_h2.T)
    b2_pad = jnp.zeros((1, 128), jnp.float32).at[0, :ncls].set(b_h2)
    out = _ktail(u1, p1, W_h1.T, b_h1[None, :], w2_pad, b2_pad)
    return out[:, :ncls].reshape(b, n, ncls)


# speedup vs baseline: 5.9908x; 1.5834x over previous
import functools
import math

import jax
import jax.numpy as jnp
from jax import lax
from jax.experimental import pallas as pl
from jax.experimental.pallas import tpu as pltpu
from jax.experimental.pallas import tpu_sc as plsc

_IT = False
_BINV = 1.0 / math.sqrt(1.0 + 1e-5)
_K = 16


def _lrelu(x):
    return jnp.where(x >= 0, x, 0.1 * x)


def _bq(x):
    return x.astype(jnp.bfloat16).astype(jnp.float32)


def _knn_body(q_ref, p_ref, i_ref, *, nk, kpad, npts):
    q = q_ref[0]
    p = p_ref[0]
    qn = jnp.sum(q * q, axis=1)[:, None]
    pn = jnp.sum(p * p, axis=1)[None, :]
    dots_b = lax.dot_general(q.astype(jnp.bfloat16), p.astype(jnp.bfloat16),
                             (((1,), (1,)), ((), ())),
                             preferred_element_type=jnp.float32)
    dsel = jnp.sqrt(jnp.maximum(qn + pn - 2.0 * dots_b, 0.0))
    r = q.shape[0]
    cols = lax.broadcasted_iota(jnp.int32, (r, npts), 1)
    ki = lax.broadcasted_iota(jnp.int32, (r, kpad), 1)
    idx_acc = jnp.zeros((r, kpad), jnp.int32)
    big = jnp.float32(3.0e38)
    for j in range(nk):
        m = jnp.min(dsel, axis=1, keepdims=True)
        cand = jnp.where(dsel <= m, cols, npts)
        im = jnp.min(cand, axis=1, keepdims=True)
        idx_acc = jnp.where(ki == j, im, idx_acc)
        if j < nk - 1:
            dsel = jnp.where(cols == im, big, dsel)
    i_ref[0] = idx_acc


def _knn(xyz_q, xyz_p, nk, row_block=256):
    b, nq, _ = xyz_q.shape
    npts = xyz_p.shape[1]
    kpad = max(nk, 8)
    body = functools.partial(_knn_body, nk=nk, kpad=kpad, npts=npts)
    return pl.pallas_call(
        body,
        grid=(b, nq // row_block),
        in_specs=[
            pl.BlockSpec((1, row_block, 3), lambda bb, i: (bb, i, 0)),
            pl.BlockSpec((1, npts, 3), lambda bb, i: (bb, 0, 0)),
        ],
        out_specs=pl.BlockSpec((1, row_block, kpad), lambda bb, i: (bb, i, 0)),
        out_shape=jax.ShapeDtypeStruct((b, nq, kpad), jnp.int32),
        interpret=_IT,
    )(xyz_q, xyz_p)


def _lfa_body(cen_ref, g_ref, nx_ref, ws_ref, wm_ref, wr_ref, o_ref):
    cen = cen_ref[...]
    g = g_ref[...]
    nx = nx_ref[...][:, :, :3]
    wr = _bq(wr_ref[...])
    diff = cen[None] - nx
    d0 = diff[:, :, 0]
    d1 = diff[:, :, 1]
    d2c = diff[:, :, 2]
    dd = d0 * d0 + d1 * d1 + d2c * d2c
    dist = jnp.sqrt(jnp.maximum(dd, 1e-20))
    kk, pp, cc = g.shape
    cenb = jnp.broadcast_to(cen[None], (kk, pp, 3))
    rels = (d0, d1, d2c, dist,
            cenb[:, :, 0], cenb[:, :, 1], cenb[:, :, 2],
            nx[:, :, 0], nx[:, :, 1], nx[:, :, 2])
    x = g
    for c, rel in enumerate(rels):
        x = x + _bq(rel)[:, :, None] * wr[c][None, None, :]
    x = _lrelu(_BINV * x)
    s = (x.reshape(kk * pp, cc) @ ws_ref[...]).reshape(kk, pp, cc)
    m = jnp.max(s, axis=0)
    e = jnp.exp(s - m[None])
    den = jnp.sum(e, axis=0)
    pooled = jnp.sum(x * e, axis=0) / den
    o_ref[...] = _lrelu(_BINV * (pooled @ wm_ref[...]))


def _lfa(cen, g3, nx3, ws_t, wm_t, wr, row_block=128):
    bn, c = g3.shape[1], g3.shape[2]
    return pl.pallas_call(
        _lfa_body,
        grid=(bn // row_block,),
        in_specs=[
            pl.BlockSpec((row_block, 3), lambda i: (i, 0)),
            pl.BlockSpec((_K, row_block, c), lambda i: (0, i, 0)),
            pl.BlockSpec((_K, row_block, 16), lambda i: (0, i, 0)),
            pl.BlockSpec((c, c), lambda i: (0, 0)),
            pl.BlockSpec((c, c), lambda i: (0, 0)),
            pl.BlockSpec((16, c), lambda i: (0, 0)),
        ],
        out_specs=pl.BlockSpec((row_block, c), lambda i: (i, 0)),
        out_shape=jax.ShapeDtypeStruct((bn, c), jnp.float32),
        interpret=_IT,
    )(cen, g3, nx3, ws_t, wm_t, wr)


def _k0_body(f_ref, wpre_ref, wf_ref, g_ref):
    x0 = _lrelu(_BINV * (f_ref[...] @ wpre_ref[...]))
    g_ref[...] = x0 @ wf_ref[...]


def _k0(feat2, wpre_t, wf_t, row_block=512):
    bn, fin = feat2.shape
    c = wf_t.shape[1]
    return pl.pallas_call(
        _k0_body,
        grid=(bn // row_block,),
        in_specs=[
            pl.BlockSpec((row_block, fin), lambda i: (i, 0)),
            pl.BlockSpec((fin, c), lambda i: (0, 0)),
            pl.BlockSpec((c, c), lambda i: (0, 0)),
        ],
        out_specs=pl.BlockSpec((row_block, c), lambda i: (i, 0)),
        out_shape=jax.ShapeDtypeStruct((bn, c), jnp.float32),
        interpret=_IT,
    )(feat2, wpre_t, wf_t)


def _kproj_body(x_ref, wf_ref, wp_ref, g_ref, p_ref):
    x = x_ref[...]
    g_ref[...] = x @ wf_ref[...]
    p_ref[...] = x @ wp_ref[...]


def _kproj(xf, wf_t, wp_t, row_block=512):
    bn, fin = xf.shape
    c = wf_t.shape[1]
    cp = wp_t.shape[1]
    return pl.pallas_call(
        _kproj_body,
        grid=(bn // row_block,),
        in_specs=[
            pl.BlockSpec((row_block, fin), lambda i: (i, 0)),
            pl.BlockSpec((fin, c), lambda i: (0, 0)),
            pl.BlockSpec((fin, cp), lambda i: (0, 0)),
        ],
        out_specs=[
            pl.BlockSpec((row_block, c), lambda i: (i, 0)),
            pl.BlockSpec((row_block, cp), lambda i: (i, 0)),
        ],
        out_shape=[
            jax.ShapeDtypeStruct((bn, c), jnp.float32),
            jax.ShapeDtypeStruct((bn, cp), jnp.float32),
        ],
        interpret=_IT,
    )(xf, wf_t, wp_t)


def _kmm_body(x_ref, w_ref, o_ref):
    o_ref[...] = x_ref[...] @ w_ref[...]


def _kmm(x, w_t, row_block=512):
    bn, fin = x.shape
    c = w_t.shape[1]
    return pl.pallas_call(
        _kmm_body,
        grid=(bn // row_block,),
        in_specs=[
            pl.BlockSpec((row_block, fin), lambda i: (i, 0)),
            pl.BlockSpec((fin, c), lambda i: (0, 0)),
        ],
        out_specs=pl.BlockSpec((row_block, c), lambda i: (i, 0)),
        out_shape=jax.ShapeDtypeStruct((bn, c), jnp.float32),
        interpret=_IT,
    )(x, w_t)


def _kmid_body(u_ref, p_ref, w_ref, o_ref):
    t = _lrelu(_BINV * (u_ref[...] + p_ref[...]))
    o_ref[...] = t @ w_ref[...]


def _kmid(u2, p2, w_t, row_block=512):
    bn, fin = u2.shape
    c = w_t.shape[1]
    return pl.pallas_call(
        _kmid_body,
        grid=(bn // row_block,),
        in_specs=[
            pl.BlockSpec((row_block, fin), lambda i: (i, 0)),
            pl.BlockSpec((row_block, fin), lambda i: (i, 0)),
            pl.BlockSpec((fin, c), lambda i: (0, 0)),
        ],
        out_specs=pl.BlockSpec((row_block, c), lambda i: (i, 0)),
        out_shape=jax.ShapeDtypeStruct((bn, c), jnp.float32),
        interpret=_IT,
    )(u2, p2, w_t)


def _ktail_body(u_ref, p_ref, w1_ref, b1_ref, w2_ref, b2_ref, o_ref):
    d1 = _lrelu(_BINV * (u_ref[...] + p_ref[...]))
    h = _lrelu(d1 @ w1_ref[...] + b1_ref[...])
    o_ref[...] = h @ w2_ref[...] + b2_ref[...]


def _ktail(u1, p1, w1_t, b1, w2_t, b2, row_block=512):
    bn, fin = u1.shape
    ch = w1_t.shape[1]
    co = w2_t.shape[1]
    return pl.pallas_call(
        _ktail_body,
        grid=(bn // row_block,),
        in_specs=[
            pl.BlockSpec((row_block, fin), lambda i: (i, 0)),
            pl.BlockSpec((row_block, fin), lambda i: (i, 0)),
            pl.BlockSpec((fin, ch), lambda i: (0, 0)),
            pl.BlockSpec((1, ch), lambda i: (0, 0)),
            pl.BlockSpec((ch, co), lambda i: (0, 0)),
            pl.BlockSpec((1, co), lambda i: (0, 0)),
        ],
        out_specs=pl.BlockSpec((row_block, co), lambda i: (i, 0)),
        out_shape=jax.ShapeDtypeStruct((bn, co), jnp.float32),
        interpret=_IT,
    )(u1, p1, w1_t, b1, w2_t, b2)


_SUB = 128


@functools.lru_cache(maxsize=None)
def _sc_gather_kernel(t, d, btot):
    nw = 32
    b_per_w = btot // nw
    ch = min(b_per_w, max(_SUB, 65536 // d))
    n_chunks = b_per_w // ch
    n_sub = ch // _SUB
    mesh = plsc.VectorSubcoreMesh(core_axis_name="c", subcore_axis_name="s")

    @functools.partial(
        pl.kernel, mesh=mesh,
        compiler_params=pltpu.CompilerParams(use_tc_tiling_on_sc=False),
        out_type=jax.ShapeDtypeStruct((btot, d), jnp.float32),
        scratch_types=[
            pltpu.VMEM((b_per_w,), jnp.int32),
            pltpu.VMEM((ch, d), jnp.float32),
            pltpu.SemaphoreType.DMA,
        ],
    )
    def k(table_hbm, idx_hbm, out_hbm, idx_v, rows_v, sem):
        wid = lax.axis_index("s") * 2 + lax.axis_index("c")
        base = wid * b_per_w
        pltpu.sync_copy(idx_hbm.at[pl.ds(base, b_per_w)], idx_v)
        for c in range(n_chunks):
            copies = [
                pltpu.async_copy(
                    table_hbm.at[idx_v.at[pl.ds(c * ch + j * _SUB, _SUB)]],
                    rows_v.at[pl.ds(j * _SUB, _SUB)], sem)
                for j in range(n_sub)
            ]
            for cp in copies:
                cp.wait()
            pltpu.sync_copy(rows_v, out_hbm.at[pl.ds(base + c * ch, ch)])

    return k


def _gather_rows(table, idx_flat):
    t, d = table.shape
    return _sc_gather_kernel(t, d, idx_flat.shape[0])(table, idx_flat)


def kernel(xyz, feat, W_pre, W1_l1, Ws_l1, Wm_l1, W1_l2, Ws_l2, Wm_l2,
           W1_l3, Ws_l3, Wm_l3, W_up2, W_up1, W_h1, b_h1, W_h2, b_h2):
    b, n, _ = xyz.shape
    nfeat = feat.shape[-1]
    ncls = W_h2.shape[0]
    m1 = max(32, -(-n // 2))
    m2 = max(32, -(-m1 // 2))
    bn = b * n

    xyz2 = xyz.reshape(bn, 3)
    xyz2p = jnp.pad(xyz2, ((0, 0), (0, 13)))
    feat2 = feat.reshape(bn, nfeat)
    offn = (jnp.arange(b, dtype=jnp.int32) * n)[:, None]
    offm1 = (jnp.arange(b, dtype=jnp.int32) * m1)[:, None]
    offm2 = (jnp.arange(b, dtype=jnp.int32) * m2)[:, None]

    def wr_pad(w1, f):
        return jnp.zeros((16, w1.shape[0]), jnp.float32).at[:10].set(
            w1[:, f:f + 10].T)

    g1t = _k0(feat2, W_pre.T, W1_l1[:, :64].T)
    n0 = _knn(xyz, xyz, _K)
    idx0t = (n0[..., :_K].reshape(b, n * _K) + offn).reshape(bn, _K).T
    idx0f = idx0t.reshape(-1)
    g1 = _gather_rows(g1t, idx0f).reshape(_K, bn, 64)
    nx0 = _gather_rows(xyz2p, idx0f).reshape(_K, bn, 16)
    x1 = _lfa(xyz2, g1, nx0, Ws_l1.T, Wm_l1.T, wr_pad(W1_l1, 64))

    sel1 = jax.random.randint(jax.random.key(1), (b, m1), 0, n)
    xyz1 = jnp.take_along_axis(xyz, sel1[:, :, None], axis=1)
    xyz1f = xyz1.reshape(b * m1, 3)
    xyz1p = jnp.pad(xyz1f, ((0, 0), (0, 13)))

    g2t, p1 = _kproj(x1, W1_l2[:, :64].T, W_up1[:, 128:].T)
    n1 = _knn(xyz1, xyz1, _K)
    n1r = n1[..., :_K].reshape(b, m1 * _K)
    comp1 = (jnp.take_along_axis(sel1, n1r, axis=1) + offn)
    idx1t = comp1.reshape(b * m1, _K).T.reshape(-1)
    nb1t = (n1r + offm1).reshape(b * m1, _K).T.reshape(-1)
    g2 = _gather_rows(g2t, idx1t).reshape(_K, b * m1, 128)
    nx1 = _gather_rows(xyz1p, nb1t).reshape(_K, b * m1, 16)
    x2 = _lfa(xyz1f, g2, nx1, Ws_l2.T, Wm_l2.T, wr_pad(W1_l2, 64))

    sel2 = jax.random.randint(jax.random.key(2), (b, m2), 0, m1)
    xyz2d = jnp.take_along_axis(xyz1, sel2[:, :, None], axis=1)
    xyz2df = xyz2d.reshape(b * m2, 3)
    xyz2dp = jnp.pad(xyz2df, ((0, 0), (0, 13)))

    g3t, p2 = _kproj(x2, W1_l3[:, :128].T, W_up2[:, 256:].T)
    n2 = _knn(xyz2d, xyz2d, _K)
    n2r = n2[..., :_K].reshape(b, m2 * _K)
    comp2 = (jnp.take_along_axis(sel2, n2r, axis=1) + offm1)
    idx2t = comp2.reshape(b * m2, _K).T.reshape(-1)
    nb2t = (n2r + offm2).reshape(b * m2, _K).T.reshape(-1)
    g3 = _gather_rows(g3t, idx2t).reshape(_K, b * m2, 256)
    nx2 = _gather_rows(xyz2dp, nb2t).reshape(_K, b * m2, 16)
    x3 = _lfa(xyz2df, g3, nx2, Ws_l3.T, Wm_l3.T, wr_pad(W1_l3, 128))

    p3 = _kmm(x3, W_up2[:, :256].T)
    nn2 = _knn(xyz1, xyz2d, 1)
    u2 = _gather_rows(p3, (nn2[..., 0] + offm2).reshape(-1))
    p2d = _kmid(u2, p2, W_up1[:, :128].T)
    nn1 = _knn(xyz, xyz1, 1)
    u1 = _gather_rows(p2d, (nn1[..., 0] + offm1).reshape(-1))

    w2_pad = jnp.zeros((64, 128), jnp.float32).at[:, :ncls].set(W_h2.T)
    b2_pad = jnp.zeros((1, 128), jnp.float32).at[0, :ncls].set(b_h2)
    out = _ktail(u1, p1, W_h1.T, b_h1[None, :], w2_pad, b2_pad)
    return out[:, :ncls].reshape(b, n, ncls)
